# baseline (speedup 1.0000x reference)
import functools

import jax
import jax.numpy as jnp
from jax import lax
from jax.experimental import pallas as pl
from jax.experimental.pallas import tpu as pltpu
from jax.experimental.pallas import tpu_sc as plsc

_NUM_UNIQ = [1000000, 1000000]
_D = 32
_F = 2

_NC = 2
_NS = 16
_NW = _NC * _NS
_LANES = 16

_SUB = 128


def _sc_gather(x_flat, table_v, rows_total, chunk, field1_off):
    per_w = rows_total // _NW
    n_chunks = per_w // chunk
    n_sub = chunk // _SUB
    n_vec = chunk // _LANES

    mesh = plsc.VectorSubcoreMesh(core_axis_name="c", subcore_axis_name="s")

    @functools.partial(
        pl.kernel,
        mesh=mesh,
        out_type=jax.ShapeDtypeStruct((rows_total, _D), jnp.float32),
        scratch_types=[
            pltpu.VMEM((chunk,), jnp.int32),
            pltpu.VMEM((chunk,), jnp.int32),
            pltpu.VMEM((chunk, 128), jnp.float32),
            pltpu.VMEM((chunk, _D), jnp.float32),
            pltpu.SemaphoreType.DMA,
        ],
        compiler_params=pltpu.CompilerParams(use_tc_tiling_on_sc=False,
                                             needs_layout_passes=False),
    )
    def gather_kernel(table_hbm, idx_hbm, out_hbm, grp_v, qoff_v, rows_v,
                      out_v, sem):
        wid = lax.axis_index("s") * _NC + lax.axis_index("c")
        base = wid * per_w
        pat = (lax.iota(jnp.int32, 16) & 1) * field1_off
        lanes = lax.iota(jnp.int32, 16)

        def chunk_body(i, carry):
            off = pl.multiple_of(base + i * chunk, _SUB)
            pltpu.sync_copy(idx_hbm.at[pl.ds(off, chunk)], grp_v)
            for j in range(n_vec):
                sl = pl.ds(j * _LANES, _LANES)
                r = grp_v[sl] + pat
                grp_v[sl] = lax.shift_right_logical(r, 2)
                qoff_v[sl] = (r & 3) * _D
            handles = []
            for j in range(n_sub):
                handles.append(
                    pltpu.async_copy(
                        table_hbm.at[grp_v.at[pl.ds(j * _SUB, _SUB)]],
                        rows_v.at[pl.ds(j * _SUB, _SUB)],
                        sem,
                    )
                )
            for h in handles:
                h.wait()
            def extract_body(j, carry2):
                i_vec = j * _LANES + lanes
                col0 = qoff_v[pl.ds(pl.multiple_of(j * _LANES, _LANES),
                                    _LANES)]
                for c in range(_D):
                    vals = plsc.load_gather(rows_v, [i_vec, col0 + c])
                    plsc.store_scatter(
                        out_v, [i_vec, jnp.full((16,), c, jnp.int32)], vals
                    )
                return carry2

            lax.fori_loop(0, n_vec, extract_body, 0)
            pltpu.sync_copy(out_v, out_hbm.at[pl.ds(off, chunk)])
            return carry

        lax.fori_loop(0, n_chunks, chunk_body, 0)

    return gather_kernel(table_v, x_flat)


def _tc_project(emb4, w4, b4, tile_n):
    n4, fd4 = emb4.shape
    d4 = w4.shape[1]

    def mm_kernel(emb_ref, w_ref, b_ref, out_ref):
        out_ref[...] = (
            jnp.dot(emb_ref[...], w_ref[...],
                    preferred_element_type=jnp.float32)
            + b_ref[...]
        )

    return pl.pallas_call(
        mm_kernel,
        grid=(n4 // tile_n,),
        in_specs=[
            pl.BlockSpec((tile_n, fd4), lambda i: (i, 0)),
            pl.BlockSpec((fd4, d4), lambda i: (0, 0)),
            pl.BlockSpec((1, d4), lambda i: (0, 0)),
        ],
        out_specs=pl.BlockSpec((tile_n, d4), lambda i: (i, 0)),
        out_shape=jax.ShapeDtypeStruct((n4, d4), jnp.float32),
    )(emb4, w4, b4)


def kernel(x, table, W, b):
    B, S, G, F = x.shape
    n_rows = B * S * G
    rows_total = n_rows * F

    x_flat = x.reshape(rows_total)
    table_v = table.reshape(table.shape[0] // 4, 4 * _D)
    gathered = _sc_gather(x_flat, table_v, rows_total, chunk=640,
                          field1_off=_NUM_UNIQ[0])
    emb4 = gathered.reshape(n_rows // 4, 4 * F * _D)
    wt = W.T
    z = jnp.zeros_like(wt)
    w4 = jnp.block([
        [wt, z, z, z],
        [z, wt, z, z],
        [z, z, wt, z],
        [z, z, z, wt],
    ])
    b4 = jnp.tile(b, 4).reshape(1, 4 * _D)
    out4 = _tc_project(emb4, w4, b4, tile_n=1024)
    return out4.reshape(B, S, G, _D)
